# baseline (speedup 1.0000x reference)
import functools

import jax
import jax.numpy as jnp
from jax import lax
from jax.experimental import pallas as pl
from jax.experimental.pallas import tpu as pltpu
from jax.experimental.pallas import tpu_sc as plsc

N_ROWS = 4096
N_COLS = 200
D = 64
B = N_ROWS * N_COLS

NC = 2
NS = 16
NW = NC * NS
B_PER_W = B // NW
CHUNK = 512
N_CHUNKS = B_PER_W // CHUNK

_mesh = plsc.VectorSubcoreMesh(core_axis_name="c", subcore_axis_name="s")


@functools.partial(
    pl.kernel,
    mesh=_mesh,
    out_type=jax.ShapeDtypeStruct((B, D), jnp.float32),
    scratch_types=[
        pltpu.VMEM((CHUNK,), jnp.int32),
        pltpu.VMEM((CHUNK, D), jnp.float32),
        pltpu.SemaphoreType.DMA,
    ],
    compiler_params=pltpu.CompilerParams(use_tc_tiling_on_sc=False),
)
def _gather_kernel(idx_hbm, table_hbm, out_hbm, idx_v, rows_v, sem):
    wid = lax.axis_index("s") * NC + lax.axis_index("c")
    base = wid * B_PER_W

    def body(i, carry):
        start = base + i * CHUNK
        pltpu.sync_copy(idx_hbm.at[pl.ds(start, CHUNK)], idx_v)
        pltpu.async_copy(table_hbm.at[idx_v], rows_v, sem).wait()
        pltpu.sync_copy(rows_v, out_hbm.at[pl.ds(start, CHUNK)])
        return carry

    lax.fori_loop(0, N_CHUNKS, body, 0)


def kernel(x, W):
    idx = x.reshape(-1).astype(jnp.int32)
    out = _gather_kernel(idx, W)
    return out.reshape(N_ROWS, N_COLS, D)


# speedup vs baseline: 1.0452x; 1.0452x over previous
import functools

import jax
import jax.numpy as jnp
from jax import lax
from jax.experimental import pallas as pl
from jax.experimental.pallas import tpu as pltpu
from jax.experimental.pallas import tpu_sc as plsc

N_ROWS = 4096
N_COLS = 200
D = 64
B = N_ROWS * N_COLS

NC = 2
NS = 16
NW = NC * NS
B_PER_W = B // NW

CHUNK = 400
K = 2
GROUP = CHUNK * K
N_GROUPS = B_PER_W // GROUP

_mesh = plsc.VectorSubcoreMesh(core_axis_name="c", subcore_axis_name="s")


@functools.partial(
    pl.kernel,
    mesh=_mesh,
    out_type=jax.ShapeDtypeStruct((B, D), jnp.float32),
    scratch_types=[
        pltpu.VMEM((B_PER_W,), jnp.int32),
        pltpu.VMEM((2, K, CHUNK, D), jnp.float32),
        pltpu.SemaphoreType.DMA,
        pltpu.SemaphoreType.DMA,
    ],
    compiler_params=pltpu.CompilerParams(use_tc_tiling_on_sc=False),
)
def _gather_kernel(idx_hbm, table_hbm, out_hbm, idx_v, rows_v, sg, sw):
    wid = lax.axis_index("s") * NC + lax.axis_index("c")
    base = wid * B_PER_W

    pltpu.sync_copy(idx_hbm.at[pl.ds(base, B_PER_W)], idx_v)

    def gather_group(t, s):
        for j in range(K):
            off = t * GROUP + j * CHUNK
            pltpu.make_async_copy(
                table_hbm.at[idx_v.at[pl.ds(off, CHUNK)]],
                rows_v.at[s, j],
                sg,
            ).start()

    def wait_gathers(s):
        for j in range(K):
            pltpu.make_async_copy(
                table_hbm.at[idx_v.at[pl.ds(0, CHUNK)]], rows_v.at[s, j], sg
            ).wait()

    def write_group(t, s):
        for j in range(K):
            off = t * GROUP + j * CHUNK
            pltpu.make_async_copy(
                rows_v.at[s, j], out_hbm.at[pl.ds(base + off, CHUNK)], sw
            ).start()

    def wait_writes(t, s):
        for j in range(K):
            off = t * GROUP + j * CHUNK
            pltpu.make_async_copy(
                rows_v.at[s, j], out_hbm.at[pl.ds(base + off, CHUNK)], sw
            ).wait()

    gather_group(0, 0)

    def body(t, carry):
        s = lax.rem(t, 2)
        wait_gathers(s)
        write_group(t, s)
        gather_group(t + 1, 1 - s)
        wait_writes(t, s)
        return carry

    lax.fori_loop(0, N_GROUPS - 1, body, 0)

    t = N_GROUPS - 1
    s = t % 2
    wait_gathers(s)
    write_group(t, s)
    wait_writes(t, s)


def kernel(x, W):
    idx = x.reshape(-1).astype(jnp.int32)
    out = _gather_kernel(idx, W)
    return out.reshape(N_ROWS, N_COLS, D)
